# baseline (speedup 1.0000x reference)
import functools

import jax
import jax.numpy as jnp
from jax import lax
from jax.experimental import pallas as pl
from jax.experimental.pallas import tpu as pltpu
from jax.experimental.pallas import tpu_sc as plsc

VOCAB = 100000
EMBED_DIM = 512
BATCH = 4096
SEQ = 50
SEQ_PAD = 56

NUM_CORES = 2
NUM_SUBCORES = 16
NUM_WORKERS = NUM_CORES * NUM_SUBCORES
TOTAL_ROWS = BATCH * SEQ_PAD
ROWS_PER_WORKER = TOTAL_ROWS // NUM_WORKERS
CHUNK = 112
NCHUNK = ROWS_PER_WORKER // CHUNK

_mesh = plsc.VectorSubcoreMesh(core_axis_name="c", subcore_axis_name="s")


@functools.partial(
    pl.kernel,
    mesh=_mesh,
    out_type=jax.ShapeDtypeStruct((TOTAL_ROWS, EMBED_DIM), jnp.float32),
    scratch_types=[
        pltpu.VMEM((ROWS_PER_WORKER,), jnp.int32),
        pltpu.VMEM((CHUNK, EMBED_DIM), jnp.float32),
        pltpu.VMEM((CHUNK, EMBED_DIM), jnp.float32),
        pltpu.SemaphoreType.DMA,
        pltpu.SemaphoreType.DMA,
    ],
)
def _embed_gather(table_hbm, idx_hbm, out_hbm, idx_v, buf0, buf1, gsem, ssem):
    wid = lax.axis_index("s") * NUM_CORES + lax.axis_index("c")
    base = wid * ROWS_PER_WORKER
    pltpu.sync_copy(idx_hbm.at[pl.ds(base, ROWS_PER_WORKER)], idx_v)
    bufs = (buf0, buf1)

    def gather(g, buf):
        return pltpu.make_async_copy(
            table_hbm.at[idx_v.at[pl.ds(g * CHUNK, CHUNK)]], buf, gsem)

    def store(g, buf):
        return pltpu.make_async_copy(
            buf, out_hbm.at[pl.ds(base + g * CHUNK, CHUNK)], ssem)

    gather(0, buf0).start()
    gather(0, buf0).wait()
    gather(1, buf1).start()
    store(0, buf0).start()

    def pair(t, carry):
        for p_off in (0, 1):
            g = 1 + 2 * t + p_off
            p = (1 + p_off) % 2
            buf, other = bufs[p], bufs[1 - p]
            gather(g, buf).wait()
            store(g - 1, other).wait()
            gather(g + 1, other).start()
            store(g, buf).start()
        return carry

    lax.fori_loop(0, (NCHUNK - 2) // 2, pair, None)

    last = NCHUNK - 1
    gather(last, bufs[last % 2]).wait()
    store(last - 1, bufs[(last - 1) % 2]).wait()
    store(last, bufs[last % 2]).start()
    store(last, bufs[last % 2]).wait()


def kernel(input_sequence, embedding_weight):
    idx = jnp.pad(input_sequence.astype(jnp.int32),
                  ((0, 0), (0, SEQ_PAD - SEQ))).reshape(-1)
    out = _embed_gather(embedding_weight, idx)
    return out.reshape(BATCH, SEQ_PAD, EMBED_DIM)[:, :SEQ, :]


# speedup vs baseline: 6.6297x; 6.6297x over previous
import functools

import jax
import jax.numpy as jnp
from jax import lax
from jax.experimental import pallas as pl
from jax.experimental.pallas import tpu as pltpu
from jax.experimental.pallas import tpu_sc as plsc

VOCAB = 100000
EMBED_DIM = 512
BATCH = 4096
SEQ = 50

NUM_CORES = 2
NUM_SUBCORES = 16
NUM_WORKERS = NUM_CORES * NUM_SUBCORES
TOTAL_ROWS = BATCH * SEQ
ROWS_PER_WORKER = TOTAL_ROWS // NUM_WORKERS
CHUNK = 80
NCHUNK = ROWS_PER_WORKER // CHUNK

_mesh = plsc.VectorSubcoreMesh(core_axis_name="c", subcore_axis_name="s")


@functools.partial(
    pl.kernel,
    mesh=_mesh,
    out_type=jax.ShapeDtypeStruct((TOTAL_ROWS, EMBED_DIM), jnp.float32),
    scratch_types=[
        pltpu.VMEM((ROWS_PER_WORKER,), jnp.int32),
        pltpu.VMEM((CHUNK, EMBED_DIM), jnp.float32),
        pltpu.VMEM((CHUNK, EMBED_DIM), jnp.float32),
        pltpu.SemaphoreType.DMA,
        pltpu.SemaphoreType.DMA,
    ],
)
def _embed_gather(table_hbm, idx_hbm, out_hbm, idx_v, buf0, buf1, gsem, ssem):
    wid = lax.axis_index("s") * NUM_CORES + lax.axis_index("c")
    base = wid * ROWS_PER_WORKER
    pltpu.sync_copy(idx_hbm.at[pl.ds(base, ROWS_PER_WORKER)], idx_v)
    bufs = (buf0, buf1)

    def gather(g, buf):
        return pltpu.make_async_copy(
            table_hbm.at[idx_v.at[pl.ds(g * CHUNK, CHUNK)]], buf, gsem)

    def store(g, buf):
        return pltpu.make_async_copy(
            buf, out_hbm.at[pl.ds(base + g * CHUNK, CHUNK)], ssem)

    gather(0, buf0).start()
    gather(0, buf0).wait()
    gather(1, buf1).start()
    store(0, buf0).start()

    def pair(t, carry):
        for p_off in (0, 1):
            g = 1 + 2 * t + p_off
            p = (1 + p_off) % 2
            buf, other = bufs[p], bufs[1 - p]
            gather(g, buf).wait()
            store(g - 1, other).wait()
            gather(g + 1, other).start()
            store(g, buf).start()
        return carry

    lax.fori_loop(0, (NCHUNK - 2) // 2, pair, None)

    last = NCHUNK - 1
    gather(last, bufs[last % 2]).wait()
    store(last - 1, bufs[(last - 1) % 2]).wait()
    store(last, bufs[last % 2]).start()
    store(last, bufs[last % 2]).wait()


def kernel(input_sequence, embedding_weight):
    idx = input_sequence.astype(jnp.int32).T.reshape(-1)
    out = _embed_gather(embedding_weight, idx)
    return out.reshape(SEQ, BATCH, EMBED_DIM).transpose(1, 0, 2)
